# speedup vs baseline: 19.5311x; 19.5311x over previous
import functools
import jax
import jax.numpy as jnp
from jax.experimental import pallas as pl
from jax.experimental.pallas import tpu as pltpu

_N = 32768
_B = 128
_K = 3277
_BOOST_STRENGTH = 1.0
_RB = 8


def _kw_body(x_ref, dc_ref, o_ref):
    x = x_ref[...]
    dc = dc_ref[...]
    bf = jnp.exp((jnp.float32(_K / _N) - dc) * jnp.float32(_BOOST_STRENGTH))
    boosted = x * bf
    ki = jax.lax.bitcast_convert_type(boosted, jnp.int32)
    key = ki ^ ((ki >> 31) & jnp.int32(0x7FFFFFFF))
    lo0 = jnp.full((_RB, 1), jnp.iinfo(jnp.int32).min, jnp.int32)
    hi0 = jnp.full((_RB, 1), jnp.iinfo(jnp.int32).max, jnp.int32)

    def body(_, carry):
        lo, hi = carry
        mid = (lo & hi) + ((lo ^ hi) >> 1)
        cnt = jnp.sum((key >= mid).astype(jnp.int32), axis=1, keepdims=True)
        pred = cnt >= _K
        return jnp.where(pred, mid, lo), jnp.where(pred, hi, mid)

    lo, _ = jax.lax.fori_loop(0, 32, body, (lo0, hi0))
    o_ref[...] = jnp.where(key >= lo, x, jnp.float32(0.0))


@jax.jit
def kernel(x, duty_cycles):
    dc2 = duty_cycles.reshape(1, _N)
    return pl.pallas_call(
        _kw_body,
        grid=(_B // _RB,),
        in_specs=[
            pl.BlockSpec((_RB, _N), lambda i: (i, 0)),
            pl.BlockSpec((1, _N), lambda i: (0, 0)),
        ],
        out_specs=pl.BlockSpec((_RB, _N), lambda i: (i, 0)),
        out_shape=jax.ShapeDtypeStruct((_B, _N), jnp.float32),
    )(x, dc2)
